# baseline (speedup 1.0000x reference)
import functools

import jax
import jax.numpy as jnp
from jax import lax
from jax.experimental import pallas as pl
from jax.experimental.pallas import tpu as pltpu
from jax.experimental.pallas import tpu_sc as plsc

B = 8
EMB = 128
H = 4
DH = EMB // H
P_N, P_E = 10000, 320000
V_N, V_E = 2048, 8192
P_MAX, V_MAX = 2048, 384
INV_SCALE = 1.0 / (H * float(DH) ** 0.5)

NP = 10240
NV = 2048
PE = 327680
VE = 8192
NC, NS = 2, 16
NW = NC * NS
CH = 128

P_NCH = PE // NW // CH
V_NCH = VE // NW // CH
SH_NP = 10112
P_RPS = SH_NP // NS
V_RPS = NV // NS
TAIL = NP - SH_NP
NBUF = 2
DEG_K = 8
P_NCH0 = 2 * (PE // NW // CH)
V_NCH1 = 2 * V_NCH

def _sc_mesh():
    return plsc.VectorSubcoreMesh(core_axis_name="c", subcore_axis_name="s",
                                  num_cores=NC, num_subcores=NS)


P_EPT = P_NCH * CH
V_EPT = V_NCH * CH


@functools.cache
def _build_sc_degrees():
    return functools.partial(
        pl.kernel,
        out_type=(
            jax.ShapeDtypeStruct((NW, NP), jnp.float32),
            jax.ShapeDtypeStruct((NW, NV), jnp.float32),
        ),
        mesh=_sc_mesh(),
        compiler_params=pltpu.CompilerParams(needs_layout_passes=False),
        scratch_types=(
            pltpu.VMEM((P_EPT,), jnp.int32),
            pltpu.VMEM((V_EPT,), jnp.int32),
            pltpu.VMEM((NP,), jnp.float32),
            pltpu.VMEM((NV,), jnp.float32),
        ),
    )(_sc_degrees_body)


def _sc_degrees(pdst, vdst):
    return _build_sc_degrees()(pdst, vdst)


def _sc_degrees_body(pdst_hbm, vdst_hbm, degp_hbm, degv_hbm,
                     pidxv, vidxv, histp, histv):
    cid = lax.axis_index("c")
    sid = lax.axis_index("s")
    wid = sid * NC + cid
    L = 16
    pltpu.sync_copy(pdst_hbm.at[pl.ds(wid * P_EPT, P_EPT)], pidxv)
    pltpu.sync_copy(vdst_hbm.at[pl.ds(wid * V_EPT, V_EPT)], vidxv)
    zeros = jnp.zeros((L,), jnp.float32)
    ones = jnp.ones((L,), jnp.float32)

    @pl.loop(0, NP, step=4 * L)
    def _zp(i):
        for k in range(4):
            histp[pl.ds(i + k * L, L)] = zeros

    @pl.loop(0, NV, step=4 * L)
    def _zv(i):
        for k in range(4):
            histv[pl.ds(i + k * L, L)] = zeros

    @pl.loop(0, P_EPT, step=4 * L)
    def _accp(i):
        for k in range(4):
            ix = pidxv[pl.ds(i + k * L, L)]
            plsc.addupdate_scatter(histp, [ix], ones)

    @pl.loop(0, V_EPT, step=4 * L)
    def _accv(i):
        for k in range(4):
            ix = vidxv[pl.ds(i + k * L, L)]
            plsc.addupdate_scatter(histv, [ix], ones)

    pltpu.sync_copy(histp, degp_hbm.at[wid])
    pltpu.sync_copy(histv, degv_hbm.at[wid])


@functools.cache
def _build_sc_segsum():
    return functools.partial(
        pl.kernel,
        out_type=(
            jax.ShapeDtypeStruct((NC, NP, EMB), jnp.float32),
            jax.ShapeDtypeStruct((NC, NV, EMB), jnp.float32),
        ),
        mesh=_sc_mesh(),
        scratch_types=(
            pltpu.VMEM_SHARED((SH_NP, EMB), jnp.float32),
            pltpu.VMEM_SHARED((NV, EMB), jnp.float32),
            [pltpu.VMEM((CH,), jnp.int32) for _ in range(NBUF)],
            [pltpu.VMEM((CH,), jnp.int32) for _ in range(NBUF)],
            [pltpu.VMEM((CH, EMB), jnp.float32) for _ in range(NBUF)],
            [pltpu.SemaphoreType.DMA for _ in range(NBUF)],
        ),
    )(_sc_segsum_body)


def _sc_segsum(yp, psrc2, pdst2, yv, vsrc2, vdst2, zeros128):
    return _build_sc_segsum()(yp, psrc2, pdst2, yv, vsrc2, vdst2, zeros128)


def _sc_segsum_body(yp_hbm, psrc_hbm, pdst_hbm, yv_hbm, vsrc_hbm, vdst_hbm,
                    zeros_hbm, zp_hbm, zv_hbm, shp, shv,
                    isrc, idst, rows, sems):
    cid = lax.axis_index("c")
    sid = lax.axis_index("s")

    def run_stream(nch, chunk_base, y_hbm, src_hbm, dst_hbm, sh):
        def e_off(j):
            return pl.multiple_of((chunk_base + j) * CH, 8)

        for b in range(NBUF):
            pltpu.sync_copy(src_hbm.at[pl.ds(e_off(b), CH)], isrc[b])
            pltpu.sync_copy(dst_hbm.at[pl.ds(e_off(b), CH)], idst[b])
            pltpu.async_copy(y_hbm.at[isrc[b]], rows[b], sems[b])

        @pl.loop(0, nch - NBUF, step=NBUF)
        def _group(g):
            for b in range(NBUF):
                j = g + b
                pltpu.make_async_copy(y_hbm.at[isrc[b]], rows[b], sems[b]).wait()
                pltpu.sync_copy(rows[b], sh.at[idst[b]], add=True)
                pltpu.sync_copy(src_hbm.at[pl.ds(e_off(j + NBUF), CH)], isrc[b])
                pltpu.sync_copy(dst_hbm.at[pl.ds(e_off(j + NBUF), CH)], idst[b])
                pltpu.async_copy(y_hbm.at[isrc[b]], rows[b], sems[b])

        for b in range(NBUF):
            pltpu.make_async_copy(y_hbm.at[isrc[b]], rows[b], sems[b]).wait()
            pltpu.sync_copy(rows[b], sh.at[idst[b]], add=True)

    pltpu.sync_copy(zeros_hbm.at[pl.ds(0, P_RPS)],
                    shp.at[pl.ds(sid * P_RPS, P_RPS)])
    pltpu.sync_copy(zeros_hbm.at[pl.ds(0, V_RPS)],
                    shv.at[pl.ds(sid * V_RPS, V_RPS)])
    plsc.subcore_barrier()

    @pl.when(cid == 0)
    def _():
        run_stream(P_NCH0, sid * P_NCH0, yp_hbm, psrc_hbm, pdst_hbm, shp)

    @pl.when(cid == 1)
    def _():
        run_stream(V_NCH1, sid * V_NCH1, yv_hbm, vsrc_hbm, vdst_hbm, shv)
    plsc.subcore_barrier()
    pltpu.sync_copy(shp.at[pl.ds(sid * P_RPS, P_RPS)],
                    zp_hbm.at[cid, pl.ds(sid * P_RPS, P_RPS)])
    @pl.when(sid == 0)
    def _():
        pltpu.sync_copy(zeros_hbm.at[pl.ds(0, TAIL)],
                        zp_hbm.at[cid, pl.ds(SH_NP, TAIL)])
    pltpu.sync_copy(shv.at[pl.ds(sid * V_RPS, V_RPS)],
                    zv_hbm.at[cid, pl.ds(sid * V_RPS, V_RPS)])


def _row_mask(i, blk, nreal):
    row = i * blk + lax.broadcasted_iota(jnp.int32, (blk, 1), 0)
    return (row < nreal).astype(jnp.float32)


def _tc_pre_body(x_ref, linW_ref, linb_ref, g1W_ref, deg_ref,
                 h0_ref, y1_ref, dinv_ref, *, blk, nreal):
    i = pl.program_id(0)
    ones_w = jnp.ones((NW, 1), jnp.float32)
    deg = lax.dot_general(deg_ref[...], ones_w, (((0,), (0,)), ((), ())),
                          preferred_element_type=jnp.float32) + 1.0
    dinv = lax.rsqrt(deg)
    m = _row_mask(i, blk, nreal)
    h0 = jnp.dot(x_ref[...], linW_ref[...],
                 preferred_element_type=jnp.float32) + linb_ref[...]
    y1 = jnp.dot(h0, g1W_ref[...], preferred_element_type=jnp.float32) * (dinv * m)
    h0_ref[...] = h0
    y1_ref[...] = y1
    dinv_ref[...] = jnp.broadcast_to(dinv, (blk, EMB))


def _tc_pre(x, linW, linb, g1W, deg, blk, nreal):
    n = x.shape[0]
    return pl.pallas_call(
        functools.partial(_tc_pre_body, blk=blk, nreal=nreal),
        grid=(n // blk,),
        in_specs=[
            pl.BlockSpec((blk, EMB), lambda i: (i, 0)),
            pl.BlockSpec((EMB, EMB), lambda i: (0, 0)),
            pl.BlockSpec((1, EMB), lambda i: (0, 0)),
            pl.BlockSpec((EMB, EMB), lambda i: (0, 0)),
            pl.BlockSpec((NW, blk), lambda i: (0, i)),
        ],
        out_specs=[pl.BlockSpec((blk, EMB), lambda i: (i, 0))] * 3,
        out_shape=[jax.ShapeDtypeStruct((n, EMB), jnp.float32)] * 3,
    )(x, linW, linb.reshape(1, EMB), g1W, deg)


def _tc_mid_body(z1_ref, y1_ref, dinv_ref, g1b_ref, g2W_ref, y2_ref, *, blk, nreal):
    i = pl.program_id(0)
    dinv = dinv_ref[...]
    m = _row_mask(i, blk, nreal)
    zsum = z1_ref[0] + z1_ref[1] + y1_ref[...]
    h1 = jnp.maximum(zsum * dinv + g1b_ref[...], 0.0)
    y2_ref[...] = jnp.dot(h1, g2W_ref[...],
                          preferred_element_type=jnp.float32) * (dinv * m)


def _tc_mid(z1, y1, dinv, g1b, g2W, blk, nreal):
    n = y1.shape[0]
    return pl.pallas_call(
        functools.partial(_tc_mid_body, blk=blk, nreal=nreal),
        grid=(n // blk,),
        in_specs=[
            pl.BlockSpec((NC, blk, EMB), lambda i: (0, i, 0)),
            pl.BlockSpec((blk, EMB), lambda i: (i, 0)),
            pl.BlockSpec((blk, EMB), lambda i: (i, 0)),
            pl.BlockSpec((1, EMB), lambda i: (0, 0)),
            pl.BlockSpec((EMB, EMB), lambda i: (0, 0)),
        ],
        out_specs=pl.BlockSpec((blk, EMB), lambda i: (i, 0)),
        out_shape=jax.ShapeDtypeStruct((n, EMB), jnp.float32),
    )(z1, y1, dinv, g1b.reshape(1, EMB), g2W)


def _tc_post_body(z2_ref, y2_ref, dinv_ref, g2b_ref, h0_ref, batch_ref,
                  wq_ref, wk_ref, a_ref, gsum_ref, *, blk, nreal, project):
    i = pl.program_id(0)
    dinv = dinv_ref[...]
    m = _row_mask(i, blk, nreal)
    zsum = z2_ref[0] + z2_ref[1] + y2_ref[...]
    h2 = (zsum * dinv + g2b_ref[...]) * m
    s = h2 + h0_ref[...] * m
    gid = lax.broadcasted_iota(jnp.int32, (B, blk), 0)
    ind = (gid == batch_ref[...]).astype(jnp.float32)
    gpart = jnp.dot(ind, h2, preferred_element_type=jnp.float32)

    @pl.when(i == 0)
    def _():
        gsum_ref[...] = jnp.zeros_like(gsum_ref)

    gsum_ref[...] += gpart
    if project:
        t = jnp.dot(s, wq_ref[...], preferred_element_type=jnp.float32)
        a_ref[...] = lax.dot_general(
            t, wk_ref[...], (((1,), (1,)), ((), ())),
            preferred_element_type=jnp.float32) * INV_SCALE
    else:
        a_ref[...] = s


def _tc_post(z2, y2, dinv, g2b, h0, batch_row, wq, wk, blk, nreal, project):
    n = y2.shape[0]
    return pl.pallas_call(
        functools.partial(_tc_post_body, blk=blk, nreal=nreal, project=project),
        grid=(n // blk,),
        in_specs=[
            pl.BlockSpec((NC, blk, EMB), lambda i: (0, i, 0)),
            pl.BlockSpec((blk, EMB), lambda i: (i, 0)),
            pl.BlockSpec((blk, EMB), lambda i: (i, 0)),
            pl.BlockSpec((1, EMB), lambda i: (0, 0)),
            pl.BlockSpec((blk, EMB), lambda i: (i, 0)),
            pl.BlockSpec((1, blk), lambda i: (0, i)),
            pl.BlockSpec((EMB, EMB), lambda i: (0, 0)),
            pl.BlockSpec((EMB, EMB), lambda i: (0, 0)),
        ],
        out_specs=[
            pl.BlockSpec((blk, EMB), lambda i: (i, 0)),
            pl.BlockSpec((B, EMB), lambda i: (0, 0)),
        ],
        out_shape=[
            jax.ShapeDtypeStruct((n, EMB), jnp.float32),
            jax.ShapeDtypeStruct((B, EMB), jnp.float32),
        ],
    )(z2, y2, dinv, g2b.reshape(1, EMB), h0, batch_row, wq, wk)


def _tc_final_body(gsp_ref, gsv_ref, cntp_ref, cntv_ref, wq_ref, wk_ref,
                   fusion_ref, rp_ref, vg_ref):
    pg = gsp_ref[...] / jnp.maximum(cntp_ref[...], 1.0)
    vg = gsv_ref[...] / jnp.maximum(cntv_ref[...], 1.0)
    fusion_ref[...] = (pg + vg) * 0.5
    vg_ref[...] = vg
    t = jnp.dot(pg, wq_ref[...], preferred_element_type=jnp.float32)
    rp_ref[...] = lax.dot_general(
        t, wk_ref[...], (((1,), (1,)), ((), ())),
        preferred_element_type=jnp.float32) * INV_SCALE


def _tc_final(gsum_p, gsum_v, cnt_p, cnt_v, wq, wk):
    return pl.pallas_call(
        _tc_final_body,
        out_shape=[jax.ShapeDtypeStruct((B, EMB), jnp.float32)] * 3,
    )(gsum_p, gsum_v, cnt_p, cnt_v, wq, wk)


PBLK = 512
NPA = NP + P_MAX
NVA = NV + V_MAX


def _tc_att_body(pstart_ref, pcnt_ref, vstart_ref, vcnt_ref,
                 a_ref, w_ref, rp_ref, vg_ref, comp_ref, mask_ref):
    b = pl.program_id(0)
    pb = pl.program_id(1)
    pc = jnp.minimum(pcnt_ref[b], P_MAX)
    vc = jnp.minimum(vcnt_ref[b], V_MAX)
    astart = pstart_ref[b] + pb * PBLK
    wstart = vstart_ref[b]
    prow = pb * PBLK + lax.broadcasted_iota(jnp.int32, (PBLK, 1), 0)
    a_blk = a_ref[pl.ds(astart, PBLK), :] * (prow < pc).astype(jnp.float32)
    a_blk = a_blk + rp_ref[0]
    vrow = lax.broadcasted_iota(jnp.int32, (V_MAX, 1), 0)
    w_blk = w_ref[pl.ds(wstart, V_MAX), :] * (vrow < vc).astype(jnp.float32)
    w_blk = w_blk + vg_ref[0]
    comp_ref[0] = lax.dot_general(
        a_blk, w_blk, (((1,), (1,)), ((), ())),
        preferred_element_type=jnp.float32)
    vm = lax.broadcasted_iota(jnp.int32, (PBLK, V_MAX), 1)
    mask_ref[0] = vm < vc


def _tc_attention(pstart, pcnt, vstart, vcnt, a_p, w_v, rp, vg):
    return pl.pallas_call(
        _tc_att_body,
        grid=(B, P_MAX // PBLK),
        in_specs=[
            pl.BlockSpec(memory_space=pltpu.SMEM),
            pl.BlockSpec(memory_space=pltpu.SMEM),
            pl.BlockSpec(memory_space=pltpu.SMEM),
            pl.BlockSpec(memory_space=pltpu.SMEM),
            pl.BlockSpec((NPA, EMB), lambda b, pb: (0, 0)),
            pl.BlockSpec((NVA, EMB), lambda b, pb: (0, 0)),
            pl.BlockSpec((1, 1, EMB), lambda b, pb: (b, 0, 0)),
            pl.BlockSpec((1, 1, EMB), lambda b, pb: (b, 0, 0)),
        ],
        out_specs=[
            pl.BlockSpec((1, PBLK, V_MAX), lambda b, pb: (b, pb, 0)),
            pl.BlockSpec((1, PBLK, V_MAX), lambda b, pb: (b, pb, 0)),
        ],
        out_shape=[
            jax.ShapeDtypeStruct((B, P_MAX, V_MAX), jnp.float32),
            jax.ShapeDtypeStruct((B, P_MAX, V_MAX), jnp.bool_),
        ],
    )(pstart, pcnt, vstart, vcnt,
      jnp.pad(a_p, ((0, NPA - NP), (0, 0))),
      jnp.pad(w_v, ((0, NVA - NV), (0, 0))),
      rp.reshape(B, 1, EMB), vg.reshape(B, 1, EMB))


def kernel(p_x, v_x, p_lin_W, p_lin_b, p_g1_W, p_g1_b, p_g2_W, p_g2_b,
           v_lin_W, v_lin_b, v_g1_W, v_g1_b, v_g2_W, v_g2_b, att_Wq, att_Wk,
           p_edge_index, p_batch, v_edge_index, v_batch):
    f32, i32 = jnp.float32, jnp.int32
    xp = jnp.pad(p_x, ((0, NP - P_N), (0, 0)))
    psrc = jnp.concatenate(
        [p_edge_index[0].astype(i32), jnp.full((PE - P_E,), P_N, i32)])
    pdst = jnp.concatenate(
        [p_edge_index[1].astype(i32), jnp.full((PE - P_E,), P_N, i32)])
    vsrc = v_edge_index[0].astype(i32)
    vdst = v_edge_index[1].astype(i32)
    pbatch_row = jnp.pad(p_batch.astype(i32), (0, NP - P_N),
                         constant_values=B).reshape(1, NP)
    vbatch_row = v_batch.astype(i32).reshape(1, NV)
    pss = jnp.searchsorted(p_batch, jnp.arange(B + 1, dtype=i32)).astype(i32)
    vss = jnp.searchsorted(v_batch, jnp.arange(B + 1, dtype=i32)).astype(i32)
    pstart, pcnt = pss[:B], pss[1:] - pss[:B]
    vstart, vcnt = vss[:B], vss[1:] - vss[:B]
    cnt_p = pcnt.astype(f32).reshape(B, 1) * jnp.ones((1, EMB), f32)
    cnt_v = vcnt.astype(f32).reshape(B, 1) * jnp.ones((1, EMB), f32)
    zeros128 = jnp.zeros((P_RPS, EMB), f32)

    degp, degv = _sc_degrees(pdst, vdst)

    h0p, y1p, dinvp = _tc_pre(xp, p_lin_W, p_lin_b, p_g1_W, degp, 512, P_N)
    h0v, y1v, dinvv = _tc_pre(v_x, v_lin_W, v_lin_b, v_g1_W, degv, 512, V_N)

    z1p, z1v = _sc_segsum(y1p, psrc, pdst, y1v, vsrc, vdst, zeros128)

    y2p = _tc_mid(z1p, y1p, dinvp, p_g1_b, p_g2_W, 512, P_N)
    y2v = _tc_mid(z1v, y1v, dinvv, v_g1_b, v_g2_W, 512, V_N)

    z2p, z2v = _sc_segsum(y2p, psrc, pdst, y2v, vsrc, vdst, zeros128)

    a_p, gsum_p = _tc_post(z2p, y2p, dinvp, p_g2_b, h0p, pbatch_row,
                           att_Wq, att_Wk, 512, P_N, True)
    w_v, gsum_v = _tc_post(z2v, y2v, dinvv, v_g2_b, h0v, vbatch_row,
                           att_Wq, att_Wk, 512, V_N, False)

    fusion, rp, vg = _tc_final(gsum_p, gsum_v, cnt_p, cnt_v, att_Wq, att_Wk)

    compatibility, att_mask = _tc_attention(
        pstart, pcnt, vstart, vcnt, a_p, w_v, rp, vg)
    return fusion, compatibility, att_mask


# speedup vs baseline: 1.3875x; 1.3875x over previous
import functools

import jax
import jax.numpy as jnp
from jax import lax
from jax.experimental import pallas as pl
from jax.experimental.pallas import tpu as pltpu
from jax.experimental.pallas import tpu_sc as plsc

B = 8
EMB = 128
H = 4
DH = EMB // H
P_N, P_E = 10000, 320000
V_N, V_E = 2048, 8192
P_MAX, V_MAX = 2048, 384
INV_SCALE = 1.0 / (H * float(DH) ** 0.5)

NP = 10240
NV = 2048
PE = 327680
VE = 8192
NC, NS = 2, 16
NW = NC * NS
CH = 128

P_NCH = PE // NW // CH
V_NCH = VE // NW // CH
SH_NP = 10112
P_RPS = SH_NP // NS
V_RPS = NV // NS
TAIL = NP - SH_NP
NBUF = 2
DEG_K = 8
P_NCH0 = 136
P_NCH1 = 2 * (PE // NW // CH) - P_NCH0
V_NCH1 = 2 * V_NCH

def _sc_mesh():
    return plsc.VectorSubcoreMesh(core_axis_name="c", subcore_axis_name="s",
                                  num_cores=NC, num_subcores=NS)


P_EPT = P_NCH * CH
V_EPT = V_NCH * CH


@functools.cache
def _build_sc_degrees():
    return functools.partial(
        pl.kernel,
        out_type=(
            jax.ShapeDtypeStruct((NW, NP), jnp.float32),
            jax.ShapeDtypeStruct((NW, NV), jnp.float32),
        ),
        mesh=_sc_mesh(),
        compiler_params=pltpu.CompilerParams(needs_layout_passes=False),
        scratch_types=(
            pltpu.VMEM((P_EPT,), jnp.int32),
            pltpu.VMEM((V_EPT,), jnp.int32),
            pltpu.VMEM((NP,), jnp.float32),
            pltpu.VMEM((NV,), jnp.float32),
        ),
    )(_sc_degrees_body)


def _sc_degrees(pdst, vdst):
    return _build_sc_degrees()(pdst, vdst)


def _sc_degrees_body(pdst_hbm, vdst_hbm, degp_hbm, degv_hbm,
                     pidxv, vidxv, histp, histv):
    cid = lax.axis_index("c")
    sid = lax.axis_index("s")
    wid = sid * NC + cid
    L = 16
    pltpu.sync_copy(pdst_hbm.at[pl.ds(wid * P_EPT, P_EPT)], pidxv)
    pltpu.sync_copy(vdst_hbm.at[pl.ds(wid * V_EPT, V_EPT)], vidxv)
    zeros = jnp.zeros((L,), jnp.float32)
    ones = jnp.ones((L,), jnp.float32)

    @pl.loop(0, NP, step=4 * L)
    def _zp(i):
        for k in range(4):
            histp[pl.ds(i + k * L, L)] = zeros

    @pl.loop(0, NV, step=4 * L)
    def _zv(i):
        for k in range(4):
            histv[pl.ds(i + k * L, L)] = zeros

    @pl.loop(0, P_EPT, step=4 * L)
    def _accp(i):
        for k in range(4):
            ix = pidxv[pl.ds(i + k * L, L)]
            plsc.addupdate_scatter(histp, [ix], ones)

    @pl.loop(0, V_EPT, step=4 * L)
    def _accv(i):
        for k in range(4):
            ix = vidxv[pl.ds(i + k * L, L)]
            plsc.addupdate_scatter(histv, [ix], ones)

    pltpu.sync_copy(histp, degp_hbm.at[wid])
    pltpu.sync_copy(histv, degv_hbm.at[wid])


@functools.cache
def _build_sc_segsum():
    return functools.partial(
        pl.kernel,
        out_type=(
            jax.ShapeDtypeStruct((NC, NP, EMB), jnp.float32),
            jax.ShapeDtypeStruct((NC, NV, EMB), jnp.float32),
        ),
        mesh=_sc_mesh(),
        scratch_types=(
            pltpu.VMEM_SHARED((SH_NP, EMB), jnp.float32),
            pltpu.VMEM_SHARED((NV, EMB), jnp.float32),
            [pltpu.VMEM((CH,), jnp.int32) for _ in range(NBUF)],
            [pltpu.VMEM((CH,), jnp.int32) for _ in range(NBUF)],
            [pltpu.VMEM((CH, EMB), jnp.float32) for _ in range(NBUF)],
            [pltpu.SemaphoreType.DMA for _ in range(NBUF)],
        ),
    )(_sc_segsum_body)


def _sc_segsum(yp, psrc2, pdst2, yv, vsrc2, vdst2, zeros128):
    return _build_sc_segsum()(yp, psrc2, pdst2, yv, vsrc2, vdst2, zeros128)


def _sc_segsum_body(yp_hbm, psrc_hbm, pdst_hbm, yv_hbm, vsrc_hbm, vdst_hbm,
                    zeros_hbm, zp_hbm, zv_hbm, shp, shv,
                    isrc, idst, rows, sems):
    cid = lax.axis_index("c")
    sid = lax.axis_index("s")

    def run_stream(nch, chunk_base, y_hbm, src_hbm, dst_hbm, sh):
        def e_off(j):
            return pl.multiple_of((chunk_base + j) * CH, 8)

        for b in range(NBUF):
            pltpu.sync_copy(src_hbm.at[pl.ds(e_off(b), CH)], isrc[b])
            pltpu.sync_copy(dst_hbm.at[pl.ds(e_off(b), CH)], idst[b])
            pltpu.async_copy(y_hbm.at[isrc[b]], rows[b], sems[b])

        @pl.loop(0, nch - NBUF, step=NBUF)
        def _group(g):
            for b in range(NBUF):
                j = g + b
                pltpu.make_async_copy(y_hbm.at[isrc[b]], rows[b], sems[b]).wait()
                pltpu.sync_copy(rows[b], sh.at[idst[b]], add=True)
                pltpu.sync_copy(src_hbm.at[pl.ds(e_off(j + NBUF), CH)], isrc[b])
                pltpu.sync_copy(dst_hbm.at[pl.ds(e_off(j + NBUF), CH)], idst[b])
                pltpu.async_copy(y_hbm.at[isrc[b]], rows[b], sems[b])

        for b in range(NBUF):
            pltpu.make_async_copy(y_hbm.at[isrc[b]], rows[b], sems[b]).wait()
            pltpu.sync_copy(rows[b], sh.at[idst[b]], add=True)

    pltpu.sync_copy(zeros_hbm.at[pl.ds(0, P_RPS)],
                    shp.at[pl.ds(sid * P_RPS, P_RPS)])
    pltpu.sync_copy(zeros_hbm.at[pl.ds(0, V_RPS)],
                    shv.at[pl.ds(sid * V_RPS, V_RPS)])
    plsc.subcore_barrier()

    @pl.when(cid == 0)
    def _():
        run_stream(P_NCH0, sid * P_NCH0, yp_hbm, psrc_hbm, pdst_hbm, shp)

    @pl.when(cid == 1)
    def _():
        run_stream(P_NCH1, NS * P_NCH0 + sid * P_NCH1,
                   yp_hbm, psrc_hbm, pdst_hbm, shp)
        run_stream(V_NCH1, sid * V_NCH1, yv_hbm, vsrc_hbm, vdst_hbm, shv)
    plsc.subcore_barrier()
    pltpu.sync_copy(shp.at[pl.ds(sid * P_RPS, P_RPS)],
                    zp_hbm.at[cid, pl.ds(sid * P_RPS, P_RPS)])
    @pl.when(sid == 0)
    def _():
        pltpu.sync_copy(zeros_hbm.at[pl.ds(0, TAIL)],
                        zp_hbm.at[cid, pl.ds(SH_NP, TAIL)])
    pltpu.sync_copy(shv.at[pl.ds(sid * V_RPS, V_RPS)],
                    zv_hbm.at[cid, pl.ds(sid * V_RPS, V_RPS)])


def _row_mask(i, blk, nreal):
    row = i * blk + lax.broadcasted_iota(jnp.int32, (blk, 1), 0)
    return (row < nreal).astype(jnp.float32)


def _tc_pre_body(x_ref, linW_ref, linb_ref, g1W_ref, deg_ref,
                 h0_ref, y1_ref, dinv_ref, *, blk, nreal):
    i = pl.program_id(0)
    ones_w = jnp.ones((NW, 1), jnp.float32)
    deg = lax.dot_general(deg_ref[...], ones_w, (((0,), (0,)), ((), ())),
                          preferred_element_type=jnp.float32) + 1.0
    dinv = lax.rsqrt(deg)
    m = _row_mask(i, blk, nreal)
    h0 = jnp.dot(x_ref[...], linW_ref[...],
                 preferred_element_type=jnp.float32) + linb_ref[...]
    y1 = jnp.dot(h0, g1W_ref[...], preferred_element_type=jnp.float32) * (dinv * m)
    h0_ref[...] = h0
    y1_ref[...] = y1
    dinv_ref[...] = jnp.broadcast_to(dinv, (blk, EMB))


def _tc_pre(x, linW, linb, g1W, deg, blk, nreal):
    n = x.shape[0]
    return pl.pallas_call(
        functools.partial(_tc_pre_body, blk=blk, nreal=nreal),
        grid=(n // blk,),
        in_specs=[
            pl.BlockSpec((blk, EMB), lambda i: (i, 0)),
            pl.BlockSpec((EMB, EMB), lambda i: (0, 0)),
            pl.BlockSpec((1, EMB), lambda i: (0, 0)),
            pl.BlockSpec((EMB, EMB), lambda i: (0, 0)),
            pl.BlockSpec((NW, blk), lambda i: (0, i)),
        ],
        out_specs=[pl.BlockSpec((blk, EMB), lambda i: (i, 0))] * 3,
        out_shape=[jax.ShapeDtypeStruct((n, EMB), jnp.float32)] * 3,
    )(x, linW, linb.reshape(1, EMB), g1W, deg)


def _tc_mid_body(z1_ref, y1_ref, dinv_ref, g1b_ref, g2W_ref, y2_ref, *, blk, nreal):
    i = pl.program_id(0)
    dinv = dinv_ref[...]
    m = _row_mask(i, blk, nreal)
    zsum = z1_ref[0] + z1_ref[1] + y1_ref[...]
    h1 = jnp.maximum(zsum * dinv + g1b_ref[...], 0.0)
    y2_ref[...] = jnp.dot(h1, g2W_ref[...],
                          preferred_element_type=jnp.float32) * (dinv * m)


def _tc_mid(z1, y1, dinv, g1b, g2W, blk, nreal):
    n = y1.shape[0]
    return pl.pallas_call(
        functools.partial(_tc_mid_body, blk=blk, nreal=nreal),
        grid=(n // blk,),
        in_specs=[
            pl.BlockSpec((NC, blk, EMB), lambda i: (0, i, 0)),
            pl.BlockSpec((blk, EMB), lambda i: (i, 0)),
            pl.BlockSpec((blk, EMB), lambda i: (i, 0)),
            pl.BlockSpec((1, EMB), lambda i: (0, 0)),
            pl.BlockSpec((EMB, EMB), lambda i: (0, 0)),
        ],
        out_specs=pl.BlockSpec((blk, EMB), lambda i: (i, 0)),
        out_shape=jax.ShapeDtypeStruct((n, EMB), jnp.float32),
    )(z1, y1, dinv, g1b.reshape(1, EMB), g2W)


def _tc_post_body(z2_ref, y2_ref, dinv_ref, g2b_ref, h0_ref, batch_ref,
                  wq_ref, wk_ref, a_ref, gsum_ref, *, blk, nreal, project):
    i = pl.program_id(0)
    dinv = dinv_ref[...]
    m = _row_mask(i, blk, nreal)
    zsum = z2_ref[0] + z2_ref[1] + y2_ref[...]
    h2 = (zsum * dinv + g2b_ref[...]) * m
    s = h2 + h0_ref[...] * m
    gid = lax.broadcasted_iota(jnp.int32, (B, blk), 0)
    ind = (gid == batch_ref[...]).astype(jnp.float32)
    gpart = jnp.dot(ind, h2, preferred_element_type=jnp.float32)

    @pl.when(i == 0)
    def _():
        gsum_ref[...] = jnp.zeros_like(gsum_ref)

    gsum_ref[...] += gpart
    if project:
        t = jnp.dot(s, wq_ref[...], preferred_element_type=jnp.float32)
        a_ref[...] = lax.dot_general(
            t, wk_ref[...], (((1,), (1,)), ((), ())),
            preferred_element_type=jnp.float32) * INV_SCALE
    else:
        a_ref[...] = s


def _tc_post(z2, y2, dinv, g2b, h0, batch_row, wq, wk, blk, nreal, project):
    n = y2.shape[0]
    return pl.pallas_call(
        functools.partial(_tc_post_body, blk=blk, nreal=nreal, project=project),
        grid=(n // blk,),
        in_specs=[
            pl.BlockSpec((NC, blk, EMB), lambda i: (0, i, 0)),
            pl.BlockSpec((blk, EMB), lambda i: (i, 0)),
            pl.BlockSpec((blk, EMB), lambda i: (i, 0)),
            pl.BlockSpec((1, EMB), lambda i: (0, 0)),
            pl.BlockSpec((blk, EMB), lambda i: (i, 0)),
            pl.BlockSpec((1, blk), lambda i: (0, i)),
            pl.BlockSpec((EMB, EMB), lambda i: (0, 0)),
            pl.BlockSpec((EMB, EMB), lambda i: (0, 0)),
        ],
        out_specs=[
            pl.BlockSpec((blk, EMB), lambda i: (i, 0)),
            pl.BlockSpec((B, EMB), lambda i: (0, 0)),
        ],
        out_shape=[
            jax.ShapeDtypeStruct((n, EMB), jnp.float32),
            jax.ShapeDtypeStruct((B, EMB), jnp.float32),
        ],
    )(z2, y2, dinv, g2b.reshape(1, EMB), h0, batch_row, wq, wk)


def _tc_final_body(gsp_ref, gsv_ref, cntp_ref, cntv_ref, wq_ref, wk_ref,
                   fusion_ref, rp_ref, vg_ref):
    pg = gsp_ref[...] / jnp.maximum(cntp_ref[...], 1.0)
    vg = gsv_ref[...] / jnp.maximum(cntv_ref[...], 1.0)
    fusion_ref[...] = (pg + vg) * 0.5
    vg_ref[...] = vg
    t = jnp.dot(pg, wq_ref[...], preferred_element_type=jnp.float32)
    rp_ref[...] = lax.dot_general(
        t, wk_ref[...], (((1,), (1,)), ((), ())),
        preferred_element_type=jnp.float32) * INV_SCALE


def _tc_final(gsum_p, gsum_v, cnt_p, cnt_v, wq, wk):
    return pl.pallas_call(
        _tc_final_body,
        out_shape=[jax.ShapeDtypeStruct((B, EMB), jnp.float32)] * 3,
    )(gsum_p, gsum_v, cnt_p, cnt_v, wq, wk)


PBLK = 512
NPA = NP + P_MAX
NVA = NV + V_MAX


def _tc_att_body(pstart_ref, pcnt_ref, vstart_ref, vcnt_ref,
                 a_ref, w_ref, rp_ref, vg_ref, comp_ref, mask_ref):
    b = pl.program_id(0)
    pb = pl.program_id(1)
    pc = jnp.minimum(pcnt_ref[b], P_MAX)
    vc = jnp.minimum(vcnt_ref[b], V_MAX)
    astart = pstart_ref[b] + pb * PBLK
    wstart = vstart_ref[b]
    prow = pb * PBLK + lax.broadcasted_iota(jnp.int32, (PBLK, 1), 0)
    a_blk = a_ref[pl.ds(astart, PBLK), :] * (prow < pc).astype(jnp.float32)
    a_blk = a_blk + rp_ref[0]
    vrow = lax.broadcasted_iota(jnp.int32, (V_MAX, 1), 0)
    w_blk = w_ref[pl.ds(wstart, V_MAX), :] * (vrow < vc).astype(jnp.float32)
    w_blk = w_blk + vg_ref[0]
    comp_ref[0] = lax.dot_general(
        a_blk, w_blk, (((1,), (1,)), ((), ())),
        preferred_element_type=jnp.float32)
    vm = lax.broadcasted_iota(jnp.int32, (PBLK, V_MAX), 1)
    mask_ref[0] = vm < vc


def _tc_attention(pstart, pcnt, vstart, vcnt, a_p, w_v, rp, vg):
    return pl.pallas_call(
        _tc_att_body,
        grid=(B, P_MAX // PBLK),
        in_specs=[
            pl.BlockSpec(memory_space=pltpu.SMEM),
            pl.BlockSpec(memory_space=pltpu.SMEM),
            pl.BlockSpec(memory_space=pltpu.SMEM),
            pl.BlockSpec(memory_space=pltpu.SMEM),
            pl.BlockSpec((NPA, EMB), lambda b, pb: (0, 0)),
            pl.BlockSpec((NVA, EMB), lambda b, pb: (0, 0)),
            pl.BlockSpec((1, 1, EMB), lambda b, pb: (b, 0, 0)),
            pl.BlockSpec((1, 1, EMB), lambda b, pb: (b, 0, 0)),
        ],
        out_specs=[
            pl.BlockSpec((1, PBLK, V_MAX), lambda b, pb: (b, pb, 0)),
            pl.BlockSpec((1, PBLK, V_MAX), lambda b, pb: (b, pb, 0)),
        ],
        out_shape=[
            jax.ShapeDtypeStruct((B, P_MAX, V_MAX), jnp.float32),
            jax.ShapeDtypeStruct((B, P_MAX, V_MAX), jnp.bool_),
        ],
    )(pstart, pcnt, vstart, vcnt,
      jnp.pad(a_p, ((0, NPA - NP), (0, 0))),
      jnp.pad(w_v, ((0, NVA - NV), (0, 0))),
      rp.reshape(B, 1, EMB), vg.reshape(B, 1, EMB))


def kernel(p_x, v_x, p_lin_W, p_lin_b, p_g1_W, p_g1_b, p_g2_W, p_g2_b,
           v_lin_W, v_lin_b, v_g1_W, v_g1_b, v_g2_W, v_g2_b, att_Wq, att_Wk,
           p_edge_index, p_batch, v_edge_index, v_batch):
    f32, i32 = jnp.float32, jnp.int32
    xp = jnp.pad(p_x, ((0, NP - P_N), (0, 0)))
    psrc = jnp.concatenate(
        [p_edge_index[0].astype(i32), jnp.full((PE - P_E,), P_N, i32)])
    pdst = jnp.concatenate(
        [p_edge_index[1].astype(i32), jnp.full((PE - P_E,), P_N, i32)])
    vsrc = v_edge_index[0].astype(i32)
    vdst = v_edge_index[1].astype(i32)
    pbatch_row = jnp.pad(p_batch.astype(i32), (0, NP - P_N),
                         constant_values=B).reshape(1, NP)
    vbatch_row = v_batch.astype(i32).reshape(1, NV)
    pss = jnp.searchsorted(p_batch, jnp.arange(B + 1, dtype=i32)).astype(i32)
    vss = jnp.searchsorted(v_batch, jnp.arange(B + 1, dtype=i32)).astype(i32)
    pstart, pcnt = pss[:B], pss[1:] - pss[:B]
    vstart, vcnt = vss[:B], vss[1:] - vss[:B]
    cnt_p = pcnt.astype(f32).reshape(B, 1) * jnp.ones((1, EMB), f32)
    cnt_v = vcnt.astype(f32).reshape(B, 1) * jnp.ones((1, EMB), f32)
    zeros128 = jnp.zeros((P_RPS, EMB), f32)

    degp, degv = _sc_degrees(pdst, vdst)

    h0p, y1p, dinvp = _tc_pre(xp, p_lin_W, p_lin_b, p_g1_W, degp, 512, P_N)
    h0v, y1v, dinvv = _tc_pre(v_x, v_lin_W, v_lin_b, v_g1_W, degv, 512, V_N)

    z1p, z1v = _sc_segsum(y1p, psrc, pdst, y1v, vsrc, vdst, zeros128)

    y2p = _tc_mid(z1p, y1p, dinvp, p_g1_b, p_g2_W, 512, P_N)
    y2v = _tc_mid(z1v, y1v, dinvv, v_g1_b, v_g2_W, 512, V_N)

    z2p, z2v = _sc_segsum(y2p, psrc, pdst, y2v, vsrc, vdst, zeros128)

    a_p, gsum_p = _tc_post(z2p, y2p, dinvp, p_g2_b, h0p, pbatch_row,
                           att_Wq, att_Wk, 512, P_N, True)
    w_v, gsum_v = _tc_post(z2v, y2v, dinvv, v_g2_b, h0v, vbatch_row,
                           att_Wq, att_Wk, 512, V_N, False)

    fusion, rp, vg = _tc_final(gsum_p, gsum_v, cnt_p, cnt_v, att_Wq, att_Wk)

    compatibility, att_mask = _tc_attention(
        pstart, pcnt, vstart, vcnt, a_p, w_v, rp, vg)
    return fusion, compatibility, att_mask


# speedup vs baseline: 1.4958x; 1.0781x over previous
import functools

import jax
import jax.numpy as jnp
from jax import lax
from jax.experimental import pallas as pl
from jax.experimental.pallas import tpu as pltpu
from jax.experimental.pallas import tpu_sc as plsc

B = 8
EMB = 128
H = 4
DH = EMB // H
P_N, P_E = 10000, 320000
V_N, V_E = 2048, 8192
P_MAX, V_MAX = 2048, 384
INV_SCALE = 1.0 / (H * float(DH) ** 0.5)

NP = 10240
NV = 2048
PE = 327680
VE = 8192
NC, NS = 2, 16
NW = NC * NS
CH = 128

P_NCH = PE // NW // CH
V_NCH = VE // NW // CH
SH_NP = 10112
P_RPS = SH_NP // NS
V_RPS = NV // NS
TAIL = NP - SH_NP
NBUF = 2
DEG_K = 8
P_NCH0 = 144
P_NCH1 = 2 * (PE // NW // CH) - P_NCH0
V_NCH1 = 2 * V_NCH

def _sc_mesh():
    return plsc.VectorSubcoreMesh(core_axis_name="c", subcore_axis_name="s",
                                  num_cores=NC, num_subcores=NS)


P_EPT = P_NCH * CH
V_EPT = V_NCH * CH


@functools.cache
def _build_sc_degrees():
    return functools.partial(
        pl.kernel,
        out_type=(
            jax.ShapeDtypeStruct((NW, NP), jnp.float32),
            jax.ShapeDtypeStruct((NW, NV), jnp.float32),
        ),
        mesh=_sc_mesh(),
        compiler_params=pltpu.CompilerParams(needs_layout_passes=False),
        scratch_types=(
            pltpu.VMEM((P_EPT,), jnp.int32),
            pltpu.VMEM((V_EPT,), jnp.int32),
            pltpu.VMEM((NP,), jnp.float32),
            pltpu.VMEM((NV,), jnp.float32),
        ),
    )(_sc_degrees_body)


def _sc_degrees(pdst, vdst):
    return _build_sc_degrees()(pdst, vdst)


def _sc_degrees_body(pdst_hbm, vdst_hbm, degp_hbm, degv_hbm,
                     pidxv, vidxv, histp, histv):
    cid = lax.axis_index("c")
    sid = lax.axis_index("s")
    wid = sid * NC + cid
    L = 16
    pltpu.sync_copy(pdst_hbm.at[pl.ds(wid * P_EPT, P_EPT)], pidxv)
    pltpu.sync_copy(vdst_hbm.at[pl.ds(wid * V_EPT, V_EPT)], vidxv)
    zeros = jnp.zeros((L,), jnp.float32)
    ones = jnp.ones((L,), jnp.float32)

    @pl.loop(0, NP, step=4 * L)
    def _zp(i):
        for k in range(4):
            histp[pl.ds(i + k * L, L)] = zeros

    @pl.loop(0, NV, step=4 * L)
    def _zv(i):
        for k in range(4):
            histv[pl.ds(i + k * L, L)] = zeros

    @pl.loop(0, P_EPT, step=4 * L)
    def _accp(i):
        for k in range(4):
            ix = pidxv[pl.ds(i + k * L, L)]
            plsc.addupdate_scatter(histp, [ix], ones)

    @pl.loop(0, V_EPT, step=4 * L)
    def _accv(i):
        for k in range(4):
            ix = vidxv[pl.ds(i + k * L, L)]
            plsc.addupdate_scatter(histv, [ix], ones)

    pltpu.sync_copy(histp, degp_hbm.at[wid])
    pltpu.sync_copy(histv, degv_hbm.at[wid])


@functools.cache
def _build_sc_segsum():
    return functools.partial(
        pl.kernel,
        out_type=(
            jax.ShapeDtypeStruct((NC, NP, EMB), jnp.float32),
            jax.ShapeDtypeStruct((NC, NV, EMB), jnp.float32),
        ),
        mesh=_sc_mesh(),
        scratch_types=(
            pltpu.VMEM_SHARED((SH_NP, EMB), jnp.float32),
            pltpu.VMEM_SHARED((NV, EMB), jnp.float32),
            [pltpu.VMEM((CH,), jnp.int32) for _ in range(NBUF)],
            [pltpu.VMEM((CH,), jnp.int32) for _ in range(NBUF)],
            [pltpu.VMEM((CH, EMB), jnp.float32) for _ in range(NBUF)],
            [pltpu.SemaphoreType.DMA for _ in range(NBUF)],
        ),
    )(_sc_segsum_body)


def _sc_segsum(yp, psrc2, pdst2, yv, vsrc2, vdst2, zeros128):
    return _build_sc_segsum()(yp, psrc2, pdst2, yv, vsrc2, vdst2, zeros128)


def _sc_segsum_body(yp_hbm, psrc_hbm, pdst_hbm, yv_hbm, vsrc_hbm, vdst_hbm,
                    zeros_hbm, zp_hbm, zv_hbm, shp, shv,
                    isrc, idst, rows, sems):
    cid = lax.axis_index("c")
    sid = lax.axis_index("s")

    def run_stream(nch, chunk_base, y_hbm, src_hbm, dst_hbm, sh):
        def e_off(j):
            return pl.multiple_of((chunk_base + j) * CH, 8)

        for b in range(NBUF):
            pltpu.sync_copy(src_hbm.at[pl.ds(e_off(b), CH)], isrc[b])
            pltpu.sync_copy(dst_hbm.at[pl.ds(e_off(b), CH)], idst[b])
            pltpu.async_copy(y_hbm.at[isrc[b]], rows[b], sems[b])

        @pl.loop(0, nch - NBUF, step=NBUF)
        def _group(g):
            for b in range(NBUF):
                j = g + b
                pltpu.make_async_copy(y_hbm.at[isrc[b]], rows[b], sems[b]).wait()
                pltpu.sync_copy(rows[b], sh.at[idst[b]], add=True)
                pltpu.sync_copy(src_hbm.at[pl.ds(e_off(j + NBUF), CH)], isrc[b])
                pltpu.sync_copy(dst_hbm.at[pl.ds(e_off(j + NBUF), CH)], idst[b])
                pltpu.async_copy(y_hbm.at[isrc[b]], rows[b], sems[b])

        for b in range(NBUF):
            pltpu.make_async_copy(y_hbm.at[isrc[b]], rows[b], sems[b]).wait()
            pltpu.sync_copy(rows[b], sh.at[idst[b]], add=True)

    pltpu.sync_copy(zeros_hbm.at[pl.ds(0, P_RPS)],
                    shp.at[pl.ds(sid * P_RPS, P_RPS)])
    pltpu.sync_copy(zeros_hbm.at[pl.ds(0, V_RPS)],
                    shv.at[pl.ds(sid * V_RPS, V_RPS)])
    plsc.subcore_barrier()

    @pl.when(cid == 0)
    def _():
        run_stream(P_NCH0, sid * P_NCH0, yp_hbm, psrc_hbm, pdst_hbm, shp)

    @pl.when(cid == 1)
    def _():
        run_stream(P_NCH1, NS * P_NCH0 + sid * P_NCH1,
                   yp_hbm, psrc_hbm, pdst_hbm, shp)
        run_stream(V_NCH1, sid * V_NCH1, yv_hbm, vsrc_hbm, vdst_hbm, shv)
    plsc.subcore_barrier()
    pltpu.sync_copy(shp.at[pl.ds(sid * P_RPS, P_RPS)],
                    zp_hbm.at[cid, pl.ds(sid * P_RPS, P_RPS)])
    @pl.when(sid == 0)
    def _():
        pltpu.sync_copy(zeros_hbm.at[pl.ds(0, TAIL)],
                        zp_hbm.at[cid, pl.ds(SH_NP, TAIL)])
    pltpu.sync_copy(shv.at[pl.ds(sid * V_RPS, V_RPS)],
                    zv_hbm.at[cid, pl.ds(sid * V_RPS, V_RPS)])


def _row_mask(i, blk, nreal):
    row = i * blk + lax.broadcasted_iota(jnp.int32, (blk, 1), 0)
    return (row < nreal).astype(jnp.float32)


def _tc_pre_body(x_ref, linW_ref, linb_ref, g1W_ref, deg_ref,
                 h0_ref, y1_ref, dinv_ref, *, blk, nreal):
    i = pl.program_id(0)
    ones_w = jnp.ones((NW, 1), jnp.float32)
    deg = lax.dot_general(deg_ref[...], ones_w, (((0,), (0,)), ((), ())),
                          preferred_element_type=jnp.float32) + 1.0
    dinv = lax.rsqrt(deg)
    m = _row_mask(i, blk, nreal)
    h0 = jnp.dot(x_ref[...], linW_ref[...],
                 preferred_element_type=jnp.float32) + linb_ref[...]
    y1 = jnp.dot(h0, g1W_ref[...], preferred_element_type=jnp.float32) * (dinv * m)
    h0_ref[...] = h0
    y1_ref[...] = y1
    dinv_ref[...] = jnp.broadcast_to(dinv, (blk, EMB))


def _tc_pre(x, linW, linb, g1W, deg, blk, nreal):
    n = x.shape[0]
    return pl.pallas_call(
        functools.partial(_tc_pre_body, blk=blk, nreal=nreal),
        grid=(n // blk,),
        in_specs=[
            pl.BlockSpec((blk, EMB), lambda i: (i, 0)),
            pl.BlockSpec((EMB, EMB), lambda i: (0, 0)),
            pl.BlockSpec((1, EMB), lambda i: (0, 0)),
            pl.BlockSpec((EMB, EMB), lambda i: (0, 0)),
            pl.BlockSpec((NW, blk), lambda i: (0, i)),
        ],
        out_specs=[pl.BlockSpec((blk, EMB), lambda i: (i, 0))] * 3,
        out_shape=[jax.ShapeDtypeStruct((n, EMB), jnp.float32)] * 3,
    )(x, linW, linb.reshape(1, EMB), g1W, deg)


def _tc_mid_body(z1_ref, y1_ref, dinv_ref, g1b_ref, g2W_ref, y2_ref, *, blk, nreal):
    i = pl.program_id(0)
    dinv = dinv_ref[...]
    m = _row_mask(i, blk, nreal)
    zsum = z1_ref[0] + z1_ref[1] + y1_ref[...]
    h1 = jnp.maximum(zsum * dinv + g1b_ref[...], 0.0)
    y2_ref[...] = jnp.dot(h1, g2W_ref[...],
                          preferred_element_type=jnp.float32) * (dinv * m)


def _tc_mid(z1, y1, dinv, g1b, g2W, blk, nreal):
    n = y1.shape[0]
    return pl.pallas_call(
        functools.partial(_tc_mid_body, blk=blk, nreal=nreal),
        grid=(n // blk,),
        in_specs=[
            pl.BlockSpec((NC, blk, EMB), lambda i: (0, i, 0)),
            pl.BlockSpec((blk, EMB), lambda i: (i, 0)),
            pl.BlockSpec((blk, EMB), lambda i: (i, 0)),
            pl.BlockSpec((1, EMB), lambda i: (0, 0)),
            pl.BlockSpec((EMB, EMB), lambda i: (0, 0)),
        ],
        out_specs=pl.BlockSpec((blk, EMB), lambda i: (i, 0)),
        out_shape=jax.ShapeDtypeStruct((n, EMB), jnp.float32),
    )(z1, y1, dinv, g1b.reshape(1, EMB), g2W)


def _tc_post_body(z2_ref, y2_ref, dinv_ref, g2b_ref, h0_ref, batch_ref,
                  wq_ref, wk_ref, a_ref, gsum_ref, *, blk, nreal, project):
    i = pl.program_id(0)
    dinv = dinv_ref[...]
    m = _row_mask(i, blk, nreal)
    zsum = z2_ref[0] + z2_ref[1] + y2_ref[...]
    h2 = (zsum * dinv + g2b_ref[...]) * m
    s = h2 + h0_ref[...] * m
    gid = lax.broadcasted_iota(jnp.int32, (B, blk), 0)
    ind = (gid == batch_ref[...]).astype(jnp.float32)
    gpart = jnp.dot(ind, h2, preferred_element_type=jnp.float32)

    @pl.when(i == 0)
    def _():
        gsum_ref[...] = jnp.zeros_like(gsum_ref)

    gsum_ref[...] += gpart
    if project:
        t = jnp.dot(s, wq_ref[...], preferred_element_type=jnp.float32)
        a_ref[...] = lax.dot_general(
            t, wk_ref[...], (((1,), (1,)), ((), ())),
            preferred_element_type=jnp.float32) * INV_SCALE
    else:
        a_ref[...] = s


def _tc_post(z2, y2, dinv, g2b, h0, batch_row, wq, wk, blk, nreal, project):
    n = y2.shape[0]
    return pl.pallas_call(
        functools.partial(_tc_post_body, blk=blk, nreal=nreal, project=project),
        grid=(n // blk,),
        in_specs=[
            pl.BlockSpec((NC, blk, EMB), lambda i: (0, i, 0)),
            pl.BlockSpec((blk, EMB), lambda i: (i, 0)),
            pl.BlockSpec((blk, EMB), lambda i: (i, 0)),
            pl.BlockSpec((1, EMB), lambda i: (0, 0)),
            pl.BlockSpec((blk, EMB), lambda i: (i, 0)),
            pl.BlockSpec((1, blk), lambda i: (0, i)),
            pl.BlockSpec((EMB, EMB), lambda i: (0, 0)),
            pl.BlockSpec((EMB, EMB), lambda i: (0, 0)),
        ],
        out_specs=[
            pl.BlockSpec((blk, EMB), lambda i: (i, 0)),
            pl.BlockSpec((B, EMB), lambda i: (0, 0)),
        ],
        out_shape=[
            jax.ShapeDtypeStruct((n, EMB), jnp.float32),
            jax.ShapeDtypeStruct((B, EMB), jnp.float32),
        ],
    )(z2, y2, dinv, g2b.reshape(1, EMB), h0, batch_row, wq, wk)


def _tc_final_body(gsp_ref, gsv_ref, cntp_ref, cntv_ref, wq_ref, wk_ref,
                   fusion_ref, rp_ref, vg_ref):
    pg = gsp_ref[...] / jnp.maximum(cntp_ref[...], 1.0)
    vg = gsv_ref[...] / jnp.maximum(cntv_ref[...], 1.0)
    fusion_ref[...] = (pg + vg) * 0.5
    vg_ref[...] = vg
    t = jnp.dot(pg, wq_ref[...], preferred_element_type=jnp.float32)
    rp_ref[...] = lax.dot_general(
        t, wk_ref[...], (((1,), (1,)), ((), ())),
        preferred_element_type=jnp.float32) * INV_SCALE


def _tc_final(gsum_p, gsum_v, cnt_p, cnt_v, wq, wk):
    return pl.pallas_call(
        _tc_final_body,
        out_shape=[jax.ShapeDtypeStruct((B, EMB), jnp.float32)] * 3,
    )(gsum_p, gsum_v, cnt_p, cnt_v, wq, wk)


PBLK = 512
NPA = NP + P_MAX
NVA = NV + V_MAX


def _tc_att_body(pstart_ref, pcnt_ref, vstart_ref, vcnt_ref,
                 a_ref, w_ref, rp_ref, vg_ref, comp_ref, mask_ref):
    b = pl.program_id(0)
    pb = pl.program_id(1)
    pc = jnp.minimum(pcnt_ref[b], P_MAX)
    vc = jnp.minimum(vcnt_ref[b], V_MAX)
    astart = pstart_ref[b] + pb * PBLK
    wstart = vstart_ref[b]
    prow = pb * PBLK + lax.broadcasted_iota(jnp.int32, (PBLK, 1), 0)
    a_blk = a_ref[pl.ds(astart, PBLK), :] * (prow < pc).astype(jnp.float32)
    a_blk = a_blk + rp_ref[0]
    vrow = lax.broadcasted_iota(jnp.int32, (V_MAX, 1), 0)
    w_blk = w_ref[pl.ds(wstart, V_MAX), :] * (vrow < vc).astype(jnp.float32)
    w_blk = w_blk + vg_ref[0]
    comp_ref[0] = lax.dot_general(
        a_blk, w_blk, (((1,), (1,)), ((), ())),
        preferred_element_type=jnp.float32)
    vm = lax.broadcasted_iota(jnp.int32, (PBLK, V_MAX), 1)
    mask_ref[0] = vm < vc


def _tc_attention(pstart, pcnt, vstart, vcnt, a_p, w_v, rp, vg):
    return pl.pallas_call(
        _tc_att_body,
        grid=(B, P_MAX // PBLK),
        in_specs=[
            pl.BlockSpec(memory_space=pltpu.SMEM),
            pl.BlockSpec(memory_space=pltpu.SMEM),
            pl.BlockSpec(memory_space=pltpu.SMEM),
            pl.BlockSpec(memory_space=pltpu.SMEM),
            pl.BlockSpec((NPA, EMB), lambda b, pb: (0, 0)),
            pl.BlockSpec((NVA, EMB), lambda b, pb: (0, 0)),
            pl.BlockSpec((1, 1, EMB), lambda b, pb: (b, 0, 0)),
            pl.BlockSpec((1, 1, EMB), lambda b, pb: (b, 0, 0)),
        ],
        out_specs=[
            pl.BlockSpec((1, PBLK, V_MAX), lambda b, pb: (b, pb, 0)),
            pl.BlockSpec((1, PBLK, V_MAX), lambda b, pb: (b, pb, 0)),
        ],
        out_shape=[
            jax.ShapeDtypeStruct((B, P_MAX, V_MAX), jnp.float32),
            jax.ShapeDtypeStruct((B, P_MAX, V_MAX), jnp.bool_),
        ],
    )(pstart, pcnt, vstart, vcnt,
      jnp.pad(a_p, ((0, NPA - NP), (0, 0))),
      jnp.pad(w_v, ((0, NVA - NV), (0, 0))),
      rp.reshape(B, 1, EMB), vg.reshape(B, 1, EMB))


def kernel(p_x, v_x, p_lin_W, p_lin_b, p_g1_W, p_g1_b, p_g2_W, p_g2_b,
           v_lin_W, v_lin_b, v_g1_W, v_g1_b, v_g2_W, v_g2_b, att_Wq, att_Wk,
           p_edge_index, p_batch, v_edge_index, v_batch):
    f32, i32 = jnp.float32, jnp.int32
    xp = jnp.pad(p_x, ((0, NP - P_N), (0, 0)))
    psrc = jnp.concatenate(
        [p_edge_index[0].astype(i32), jnp.full((PE - P_E,), P_N, i32)])
    pdst = jnp.concatenate(
        [p_edge_index[1].astype(i32), jnp.full((PE - P_E,), P_N, i32)])
    vsrc = v_edge_index[0].astype(i32)
    vdst = v_edge_index[1].astype(i32)
    pbatch_row = jnp.pad(p_batch.astype(i32), (0, NP - P_N),
                         constant_values=B).reshape(1, NP)
    vbatch_row = v_batch.astype(i32).reshape(1, NV)
    pss = jnp.searchsorted(p_batch, jnp.arange(B + 1, dtype=i32)).astype(i32)
    vss = jnp.searchsorted(v_batch, jnp.arange(B + 1, dtype=i32)).astype(i32)
    pstart, pcnt = pss[:B], pss[1:] - pss[:B]
    vstart, vcnt = vss[:B], vss[1:] - vss[:B]
    cnt_p = pcnt.astype(f32).reshape(B, 1) * jnp.ones((1, EMB), f32)
    cnt_v = vcnt.astype(f32).reshape(B, 1) * jnp.ones((1, EMB), f32)
    zeros128 = jnp.zeros((P_RPS, EMB), f32)

    degp, degv = _sc_degrees(pdst, vdst)

    h0p, y1p, dinvp = _tc_pre(xp, p_lin_W, p_lin_b, p_g1_W, degp, 512, P_N)
    h0v, y1v, dinvv = _tc_pre(v_x, v_lin_W, v_lin_b, v_g1_W, degv, 512, V_N)

    z1p, z1v = _sc_segsum(y1p, psrc, pdst, y1v, vsrc, vdst, zeros128)

    y2p = _tc_mid(z1p, y1p, dinvp, p_g1_b, p_g2_W, 512, P_N)
    y2v = _tc_mid(z1v, y1v, dinvv, v_g1_b, v_g2_W, 512, V_N)

    z2p, z2v = _sc_segsum(y2p, psrc, pdst, y2v, vsrc, vdst, zeros128)

    a_p, gsum_p = _tc_post(z2p, y2p, dinvp, p_g2_b, h0p, pbatch_row,
                           att_Wq, att_Wk, 512, P_N, True)
    w_v, gsum_v = _tc_post(z2v, y2v, dinvv, v_g2_b, h0v, vbatch_row,
                           att_Wq, att_Wk, 512, V_N, False)

    fusion, rp, vg = _tc_final(gsum_p, gsum_v, cnt_p, cnt_v, att_Wq, att_Wk)

    compatibility, att_mask = _tc_attention(
        pstart, pcnt, vstart, vcnt, a_p, w_v, rp, vg)
    return fusion, compatibility, att_mask


# speedup vs baseline: 1.5167x; 1.0140x over previous
import functools

import jax
import jax.numpy as jnp
from jax import lax
from jax.experimental import pallas as pl
from jax.experimental.pallas import tpu as pltpu
from jax.experimental.pallas import tpu_sc as plsc

B = 8
EMB = 128
H = 4
DH = EMB // H
P_N, P_E = 10000, 320000
V_N, V_E = 2048, 8192
P_MAX, V_MAX = 2048, 384
INV_SCALE = 1.0 / (H * float(DH) ** 0.5)

NP = 10240
NV = 2048
PE = 327680
VE = 8192
NC, NS = 2, 16
NW = NC * NS
CH = 128

P_NCH = PE // NW // CH
V_NCH = VE // NW // CH
SH_NP = 10112
P_RPS = SH_NP // NS
V_RPS = NV // NS
TAIL = NP - SH_NP
NBUF = 2
DEG_K = 8
P_NCH0 = 152
P_NCH1 = 2 * (PE // NW // CH) - P_NCH0
V_NCH1 = 2 * V_NCH

def _sc_mesh():
    return plsc.VectorSubcoreMesh(core_axis_name="c", subcore_axis_name="s",
                                  num_cores=NC, num_subcores=NS)


P_EPT = P_NCH * CH
V_EPT = V_NCH * CH


@functools.cache
def _build_sc_degrees():
    return functools.partial(
        pl.kernel,
        out_type=(
            jax.ShapeDtypeStruct((NW, NP), jnp.float32),
            jax.ShapeDtypeStruct((NW, NV), jnp.float32),
        ),
        mesh=_sc_mesh(),
        compiler_params=pltpu.CompilerParams(needs_layout_passes=False),
        scratch_types=(
            pltpu.VMEM((P_EPT,), jnp.int32),
            pltpu.VMEM((V_EPT,), jnp.int32),
            pltpu.VMEM((NP,), jnp.float32),
            pltpu.VMEM((NV,), jnp.float32),
        ),
    )(_sc_degrees_body)


def _sc_degrees(pdst, vdst):
    return _build_sc_degrees()(pdst, vdst)


def _sc_degrees_body(pdst_hbm, vdst_hbm, degp_hbm, degv_hbm,
                     pidxv, vidxv, histp, histv):
    cid = lax.axis_index("c")
    sid = lax.axis_index("s")
    wid = sid * NC + cid
    L = 16
    pltpu.sync_copy(pdst_hbm.at[pl.ds(wid * P_EPT, P_EPT)], pidxv)
    pltpu.sync_copy(vdst_hbm.at[pl.ds(wid * V_EPT, V_EPT)], vidxv)
    zeros = jnp.zeros((L,), jnp.float32)
    ones = jnp.ones((L,), jnp.float32)

    @pl.loop(0, NP, step=4 * L)
    def _zp(i):
        for k in range(4):
            histp[pl.ds(i + k * L, L)] = zeros

    @pl.loop(0, NV, step=4 * L)
    def _zv(i):
        for k in range(4):
            histv[pl.ds(i + k * L, L)] = zeros

    @pl.loop(0, P_EPT, step=4 * L)
    def _accp(i):
        for k in range(4):
            ix = pidxv[pl.ds(i + k * L, L)]
            plsc.addupdate_scatter(histp, [ix], ones)

    @pl.loop(0, V_EPT, step=4 * L)
    def _accv(i):
        for k in range(4):
            ix = vidxv[pl.ds(i + k * L, L)]
            plsc.addupdate_scatter(histv, [ix], ones)

    pltpu.sync_copy(histp, degp_hbm.at[wid])
    pltpu.sync_copy(histv, degv_hbm.at[wid])


@functools.cache
def _build_sc_segsum():
    return functools.partial(
        pl.kernel,
        out_type=(
            jax.ShapeDtypeStruct((NC, NP, EMB), jnp.float32),
            jax.ShapeDtypeStruct((NC, NV, EMB), jnp.float32),
        ),
        mesh=_sc_mesh(),
        scratch_types=(
            pltpu.VMEM_SHARED((SH_NP, EMB), jnp.float32),
            pltpu.VMEM_SHARED((NV, EMB), jnp.float32),
            [pltpu.VMEM((CH,), jnp.int32) for _ in range(NBUF)],
            [pltpu.VMEM((CH,), jnp.int32) for _ in range(NBUF)],
            [pltpu.VMEM((CH, EMB), jnp.float32) for _ in range(NBUF)],
            [pltpu.SemaphoreType.DMA for _ in range(NBUF)],
        ),
    )(_sc_segsum_body)


def _sc_segsum(yp, psrc2, pdst2, yv, vsrc2, vdst2, zeros128):
    return _build_sc_segsum()(yp, psrc2, pdst2, yv, vsrc2, vdst2, zeros128)


def _sc_segsum_body(yp_hbm, psrc_hbm, pdst_hbm, yv_hbm, vsrc_hbm, vdst_hbm,
                    zeros_hbm, zp_hbm, zv_hbm, shp, shv,
                    isrc, idst, rows, sems):
    cid = lax.axis_index("c")
    sid = lax.axis_index("s")

    def run_stream(nch, chunk_base, y_hbm, src_hbm, dst_hbm, sh):
        def e_off(j):
            return pl.multiple_of((chunk_base + j) * CH, 8)

        for b in range(NBUF):
            pltpu.sync_copy(src_hbm.at[pl.ds(e_off(b), CH)], isrc[b])
            pltpu.sync_copy(dst_hbm.at[pl.ds(e_off(b), CH)], idst[b])
            pltpu.async_copy(y_hbm.at[isrc[b]], rows[b], sems[b])

        @pl.loop(0, nch - NBUF, step=NBUF)
        def _group(g):
            for b in range(NBUF):
                j = g + b
                pltpu.make_async_copy(y_hbm.at[isrc[b]], rows[b], sems[b]).wait()
                pltpu.sync_copy(rows[b], sh.at[idst[b]], add=True)
                pltpu.sync_copy(src_hbm.at[pl.ds(e_off(j + NBUF), CH)], isrc[b])
                pltpu.sync_copy(dst_hbm.at[pl.ds(e_off(j + NBUF), CH)], idst[b])
                pltpu.async_copy(y_hbm.at[isrc[b]], rows[b], sems[b])

        for b in range(NBUF):
            pltpu.make_async_copy(y_hbm.at[isrc[b]], rows[b], sems[b]).wait()
            pltpu.sync_copy(rows[b], sh.at[idst[b]], add=True)

    pltpu.sync_copy(zeros_hbm.at[pl.ds(0, P_RPS)],
                    shp.at[pl.ds(sid * P_RPS, P_RPS)])
    pltpu.sync_copy(zeros_hbm.at[pl.ds(0, V_RPS)],
                    shv.at[pl.ds(sid * V_RPS, V_RPS)])
    plsc.subcore_barrier()

    @pl.when(cid == 0)
    def _():
        run_stream(P_NCH0, sid * P_NCH0, yp_hbm, psrc_hbm, pdst_hbm, shp)

    @pl.when(cid == 1)
    def _():
        run_stream(P_NCH1, NS * P_NCH0 + sid * P_NCH1,
                   yp_hbm, psrc_hbm, pdst_hbm, shp)
        run_stream(V_NCH1, sid * V_NCH1, yv_hbm, vsrc_hbm, vdst_hbm, shv)
    plsc.subcore_barrier()
    pltpu.sync_copy(shp.at[pl.ds(sid * P_RPS, P_RPS)],
                    zp_hbm.at[cid, pl.ds(sid * P_RPS, P_RPS)])
    @pl.when(sid == 0)
    def _():
        pltpu.sync_copy(zeros_hbm.at[pl.ds(0, TAIL)],
                        zp_hbm.at[cid, pl.ds(SH_NP, TAIL)])
    pltpu.sync_copy(shv.at[pl.ds(sid * V_RPS, V_RPS)],
                    zv_hbm.at[cid, pl.ds(sid * V_RPS, V_RPS)])


def _row_mask(i, blk, nreal):
    row = i * blk + lax.broadcasted_iota(jnp.int32, (blk, 1), 0)
    return (row < nreal).astype(jnp.float32)


def _tc_pre_body(x_ref, linW_ref, linb_ref, g1W_ref, deg_ref,
                 h0_ref, y1_ref, dinv_ref, *, blk, nreal):
    i = pl.program_id(0)
    ones_w = jnp.ones((NW, 1), jnp.float32)
    deg = lax.dot_general(deg_ref[...], ones_w, (((0,), (0,)), ((), ())),
                          preferred_element_type=jnp.float32) + 1.0
    dinv = lax.rsqrt(deg)
    m = _row_mask(i, blk, nreal)
    h0 = jnp.dot(x_ref[...], linW_ref[...],
                 preferred_element_type=jnp.float32) + linb_ref[...]
    y1 = jnp.dot(h0, g1W_ref[...], preferred_element_type=jnp.float32) * (dinv * m)
    h0_ref[...] = h0
    y1_ref[...] = y1
    dinv_ref[...] = jnp.broadcast_to(dinv, (blk, EMB))


def _tc_pre(x, linW, linb, g1W, deg, blk, nreal):
    n = x.shape[0]
    return pl.pallas_call(
        functools.partial(_tc_pre_body, blk=blk, nreal=nreal),
        grid=(n // blk,),
        in_specs=[
            pl.BlockSpec((blk, EMB), lambda i: (i, 0)),
            pl.BlockSpec((EMB, EMB), lambda i: (0, 0)),
            pl.BlockSpec((1, EMB), lambda i: (0, 0)),
            pl.BlockSpec((EMB, EMB), lambda i: (0, 0)),
            pl.BlockSpec((NW, blk), lambda i: (0, i)),
        ],
        out_specs=[pl.BlockSpec((blk, EMB), lambda i: (i, 0))] * 3,
        out_shape=[jax.ShapeDtypeStruct((n, EMB), jnp.float32)] * 3,
    )(x, linW, linb.reshape(1, EMB), g1W, deg)


def _tc_mid_body(z1_ref, y1_ref, dinv_ref, g1b_ref, g2W_ref, y2_ref, *, blk, nreal):
    i = pl.program_id(0)
    dinv = dinv_ref[...]
    m = _row_mask(i, blk, nreal)
    zsum = z1_ref[0] + z1_ref[1] + y1_ref[...]
    h1 = jnp.maximum(zsum * dinv + g1b_ref[...], 0.0)
    y2_ref[...] = jnp.dot(h1, g2W_ref[...],
                          preferred_element_type=jnp.float32) * (dinv * m)


def _tc_mid(z1, y1, dinv, g1b, g2W, blk, nreal):
    n = y1.shape[0]
    return pl.pallas_call(
        functools.partial(_tc_mid_body, blk=blk, nreal=nreal),
        grid=(n // blk,),
        in_specs=[
            pl.BlockSpec((NC, blk, EMB), lambda i: (0, i, 0)),
            pl.BlockSpec((blk, EMB), lambda i: (i, 0)),
            pl.BlockSpec((blk, EMB), lambda i: (i, 0)),
            pl.BlockSpec((1, EMB), lambda i: (0, 0)),
            pl.BlockSpec((EMB, EMB), lambda i: (0, 0)),
        ],
        out_specs=pl.BlockSpec((blk, EMB), lambda i: (i, 0)),
        out_shape=jax.ShapeDtypeStruct((n, EMB), jnp.float32),
    )(z1, y1, dinv, g1b.reshape(1, EMB), g2W)


def _tc_post_body(z2_ref, y2_ref, dinv_ref, g2b_ref, h0_ref, batch_ref,
                  wq_ref, wk_ref, a_ref, gsum_ref, *, blk, nreal, project):
    i = pl.program_id(0)
    dinv = dinv_ref[...]
    m = _row_mask(i, blk, nreal)
    zsum = z2_ref[0] + z2_ref[1] + y2_ref[...]
    h2 = (zsum * dinv + g2b_ref[...]) * m
    s = h2 + h0_ref[...] * m
    gid = lax.broadcasted_iota(jnp.int32, (B, blk), 0)
    ind = (gid == batch_ref[...]).astype(jnp.float32)
    gpart = jnp.dot(ind, h2, preferred_element_type=jnp.float32)

    @pl.when(i == 0)
    def _():
        gsum_ref[...] = jnp.zeros_like(gsum_ref)

    gsum_ref[...] += gpart
    if project:
        t = jnp.dot(s, wq_ref[...], preferred_element_type=jnp.float32)
        a_ref[...] = lax.dot_general(
            t, wk_ref[...], (((1,), (1,)), ((), ())),
            preferred_element_type=jnp.float32) * INV_SCALE
    else:
        a_ref[...] = s


def _tc_post(z2, y2, dinv, g2b, h0, batch_row, wq, wk, blk, nreal, project):
    n = y2.shape[0]
    return pl.pallas_call(
        functools.partial(_tc_post_body, blk=blk, nreal=nreal, project=project),
        grid=(n // blk,),
        in_specs=[
            pl.BlockSpec((NC, blk, EMB), lambda i: (0, i, 0)),
            pl.BlockSpec((blk, EMB), lambda i: (i, 0)),
            pl.BlockSpec((blk, EMB), lambda i: (i, 0)),
            pl.BlockSpec((1, EMB), lambda i: (0, 0)),
            pl.BlockSpec((blk, EMB), lambda i: (i, 0)),
            pl.BlockSpec((1, blk), lambda i: (0, i)),
            pl.BlockSpec((EMB, EMB), lambda i: (0, 0)),
            pl.BlockSpec((EMB, EMB), lambda i: (0, 0)),
        ],
        out_specs=[
            pl.BlockSpec((blk, EMB), lambda i: (i, 0)),
            pl.BlockSpec((B, EMB), lambda i: (0, 0)),
        ],
        out_shape=[
            jax.ShapeDtypeStruct((n, EMB), jnp.float32),
            jax.ShapeDtypeStruct((B, EMB), jnp.float32),
        ],
    )(z2, y2, dinv, g2b.reshape(1, EMB), h0, batch_row, wq, wk)


def _tc_final_body(gsp_ref, gsv_ref, cntp_ref, cntv_ref, wq_ref, wk_ref,
                   fusion_ref, rp_ref, vg_ref):
    pg = gsp_ref[...] / jnp.maximum(cntp_ref[...], 1.0)
    vg = gsv_ref[...] / jnp.maximum(cntv_ref[...], 1.0)
    fusion_ref[...] = (pg + vg) * 0.5
    vg_ref[...] = vg
    t = jnp.dot(pg, wq_ref[...], preferred_element_type=jnp.float32)
    rp_ref[...] = lax.dot_general(
        t, wk_ref[...], (((1,), (1,)), ((), ())),
        preferred_element_type=jnp.float32) * INV_SCALE


def _tc_final(gsum_p, gsum_v, cnt_p, cnt_v, wq, wk):
    return pl.pallas_call(
        _tc_final_body,
        out_shape=[jax.ShapeDtypeStruct((B, EMB), jnp.float32)] * 3,
    )(gsum_p, gsum_v, cnt_p, cnt_v, wq, wk)


PBLK = 512
NPA = NP + P_MAX
NVA = NV + V_MAX


def _tc_att_body(pstart_ref, pcnt_ref, vstart_ref, vcnt_ref,
                 a_ref, w_ref, rp_ref, vg_ref, comp_ref, mask_ref):
    b = pl.program_id(0)
    pb = pl.program_id(1)
    pc = jnp.minimum(pcnt_ref[b], P_MAX)
    vc = jnp.minimum(vcnt_ref[b], V_MAX)
    astart = pstart_ref[b] + pb * PBLK
    wstart = vstart_ref[b]
    prow = pb * PBLK + lax.broadcasted_iota(jnp.int32, (PBLK, 1), 0)
    a_blk = a_ref[pl.ds(astart, PBLK), :] * (prow < pc).astype(jnp.float32)
    a_blk = a_blk + rp_ref[0]
    vrow = lax.broadcasted_iota(jnp.int32, (V_MAX, 1), 0)
    w_blk = w_ref[pl.ds(wstart, V_MAX), :] * (vrow < vc).astype(jnp.float32)
    w_blk = w_blk + vg_ref[0]
    comp_ref[0] = lax.dot_general(
        a_blk, w_blk, (((1,), (1,)), ((), ())),
        preferred_element_type=jnp.float32)
    vm = lax.broadcasted_iota(jnp.int32, (PBLK, V_MAX), 1)
    mask_ref[0] = vm < vc


def _tc_attention(pstart, pcnt, vstart, vcnt, a_p, w_v, rp, vg):
    return pl.pallas_call(
        _tc_att_body,
        grid=(B, P_MAX // PBLK),
        in_specs=[
            pl.BlockSpec(memory_space=pltpu.SMEM),
            pl.BlockSpec(memory_space=pltpu.SMEM),
            pl.BlockSpec(memory_space=pltpu.SMEM),
            pl.BlockSpec(memory_space=pltpu.SMEM),
            pl.BlockSpec((NPA, EMB), lambda b, pb: (0, 0)),
            pl.BlockSpec((NVA, EMB), lambda b, pb: (0, 0)),
            pl.BlockSpec((1, 1, EMB), lambda b, pb: (b, 0, 0)),
            pl.BlockSpec((1, 1, EMB), lambda b, pb: (b, 0, 0)),
        ],
        out_specs=[
            pl.BlockSpec((1, PBLK, V_MAX), lambda b, pb: (b, pb, 0)),
            pl.BlockSpec((1, PBLK, V_MAX), lambda b, pb: (b, pb, 0)),
        ],
        out_shape=[
            jax.ShapeDtypeStruct((B, P_MAX, V_MAX), jnp.float32),
            jax.ShapeDtypeStruct((B, P_MAX, V_MAX), jnp.bool_),
        ],
    )(pstart, pcnt, vstart, vcnt,
      jnp.pad(a_p, ((0, NPA - NP), (0, 0))),
      jnp.pad(w_v, ((0, NVA - NV), (0, 0))),
      rp.reshape(B, 1, EMB), vg.reshape(B, 1, EMB))


def kernel(p_x, v_x, p_lin_W, p_lin_b, p_g1_W, p_g1_b, p_g2_W, p_g2_b,
           v_lin_W, v_lin_b, v_g1_W, v_g1_b, v_g2_W, v_g2_b, att_Wq, att_Wk,
           p_edge_index, p_batch, v_edge_index, v_batch):
    f32, i32 = jnp.float32, jnp.int32
    xp = jnp.pad(p_x, ((0, NP - P_N), (0, 0)))
    psrc = jnp.concatenate(
        [p_edge_index[0].astype(i32), jnp.full((PE - P_E,), P_N, i32)])
    pdst = jnp.concatenate(
        [p_edge_index[1].astype(i32), jnp.full((PE - P_E,), P_N, i32)])
    vsrc = v_edge_index[0].astype(i32)
    vdst = v_edge_index[1].astype(i32)
    pbatch_row = jnp.pad(p_batch.astype(i32), (0, NP - P_N),
                         constant_values=B).reshape(1, NP)
    vbatch_row = v_batch.astype(i32).reshape(1, NV)
    pss = jnp.searchsorted(p_batch, jnp.arange(B + 1, dtype=i32)).astype(i32)
    vss = jnp.searchsorted(v_batch, jnp.arange(B + 1, dtype=i32)).astype(i32)
    pstart, pcnt = pss[:B], pss[1:] - pss[:B]
    vstart, vcnt = vss[:B], vss[1:] - vss[:B]
    cnt_p = pcnt.astype(f32).reshape(B, 1) * jnp.ones((1, EMB), f32)
    cnt_v = vcnt.astype(f32).reshape(B, 1) * jnp.ones((1, EMB), f32)
    zeros128 = jnp.zeros((P_RPS, EMB), f32)

    degp, degv = _sc_degrees(pdst, vdst)

    h0p, y1p, dinvp = _tc_pre(xp, p_lin_W, p_lin_b, p_g1_W, degp, 512, P_N)
    h0v, y1v, dinvv = _tc_pre(v_x, v_lin_W, v_lin_b, v_g1_W, degv, 512, V_N)

    z1p, z1v = _sc_segsum(y1p, psrc, pdst, y1v, vsrc, vdst, zeros128)

    y2p = _tc_mid(z1p, y1p, dinvp, p_g1_b, p_g2_W, 512, P_N)
    y2v = _tc_mid(z1v, y1v, dinvv, v_g1_b, v_g2_W, 512, V_N)

    z2p, z2v = _sc_segsum(y2p, psrc, pdst, y2v, vsrc, vdst, zeros128)

    a_p, gsum_p = _tc_post(z2p, y2p, dinvp, p_g2_b, h0p, pbatch_row,
                           att_Wq, att_Wk, 512, P_N, True)
    w_v, gsum_v = _tc_post(z2v, y2v, dinvv, v_g2_b, h0v, vbatch_row,
                           att_Wq, att_Wk, 512, V_N, False)

    fusion, rp, vg = _tc_final(gsum_p, gsum_v, cnt_p, cnt_v, att_Wq, att_Wk)

    compatibility, att_mask = _tc_attention(
        pstart, pcnt, vstart, vcnt, a_p, w_v, rp, vg)
    return fusion, compatibility, att_mask
